# baseline (speedup 1.0000x reference)
import functools

import jax
import jax.numpy as jnp
from jax import lax
from jax.experimental import pallas as pl
from jax.experimental.pallas import tpu as pltpu
from jax.experimental.pallas import tpu_sc as plsc

N, E, D, DE, H = 10000, 320000, 128, 16, 4
HD = H * D
NC, NS, LL = 2, 16, 16
NW = NC * NS
CHUNK = 80
PER_W = E // NW
NCH_W = PER_W // CHUNK
PER_T = E // NS
NCH_T = PER_T // CHUNK
ROWS_T = 1000
EB = 512
NEB = E // EB
NB = 2000
NNB = N // NB

_mesh = plsc.VectorSubcoreMesh(core_axis_name="c", subcore_axis_name="s",
                               num_cores=NC, num_subcores=NS)


def _sp(x):
    return jnp.maximum(x, 0.0) + jnp.log1p(jnp.exp(-jnp.abs(x)))


def _silu(x):
    return x * (1.0 / (1.0 + jnp.exp(-x)))


def _alpha_body(ai_ref, aj_ref, ef_ref, wx_ref, we_ref,
                att1_ref, att2_ref, g2_ref, beta_ref, o_ref):
    et = jnp.dot(ef_ref[...], we_ref[...], preferred_element_type=jnp.float32)
    xi = _sp(jnp.dot(ai_ref[...].astype(jnp.bfloat16), wx_ref[...],
                     preferred_element_type=jnp.float32) + et)
    xj = _sp(jnp.dot(aj_ref[...].astype(jnp.bfloat16), wx_ref[...],
                     preferred_element_type=jnp.float32) + et)
    cols = []
    for h in range(H):
        a1 = att1_ref[h:h + 1, :]
        a2 = att2_ref[h:h + 1, :]
        d = (jnp.sum(xi[:, h * D:(h + 1) * D] * a1, axis=1, keepdims=True)
             + jnp.sum(xj[:, h * D:(h + 1) * D] * a2, axis=1, keepdims=True))
        cols.append(d)
    cols.append(jnp.zeros((EB, 16 - H), jnp.float32))
    draw = jnp.concatenate(cols, axis=1)
    alpha = _sp(_sp(draw) * g2_ref[...] + beta_ref[...])
    o_ref[...] = alpha


def _tc_alpha(A_i, A_j, edge_fea, Wx_bf, We, att1, att2, g2, beta):
    return pl.pallas_call(
        _alpha_body,
        grid=(NEB,),
        in_specs=[pl.BlockSpec((EB, D), lambda b: (b, 0)),
                  pl.BlockSpec((EB, D), lambda b: (b, 0)),
                  pl.BlockSpec((EB, DE), lambda b: (b, 0)),
                  pl.BlockSpec((D, HD), lambda b: (0, 0)),
                  pl.BlockSpec((DE, HD), lambda b: (0, 0)),
                  pl.BlockSpec((H, D), lambda b: (0, 0)),
                  pl.BlockSpec((H, D), lambda b: (0, 0)),
                  pl.BlockSpec((1, 16), lambda b: (0, 0)),
                  pl.BlockSpec((1, 16), lambda b: (0, 0))],
        out_specs=pl.BlockSpec((EB, 16), lambda b: (b, 0)),
        out_shape=jax.ShapeDtypeStruct((E, 16), jnp.float32),
    )(A_i, A_j, edge_fea, Wx_bf, We, att1, att2, g2, beta)


def _msum_body(aj_ref, ef_ref, af_ref, wx_ref, we_ref, o_ref):
    et = jnp.dot(ef_ref[...], we_ref[...], preferred_element_type=jnp.float32)
    xj = _sp(jnp.dot(aj_ref[...].astype(jnp.bfloat16), wx_ref[...],
                     preferred_element_type=jnp.float32) + et)
    af = af_ref[...]
    acc = xj[:, 0:D] * af[:, 0:1]
    for h in range(1, H):
        acc = acc + xj[:, h * D:(h + 1) * D] * af[:, h:h + 1]
    o_ref[...] = acc * (1.0 / H)


def _tc_msum(A_j, edge_fea, alphaf, Wx_bf, We):
    return pl.pallas_call(
        _msum_body,
        grid=(NEB,),
        in_specs=[pl.BlockSpec((EB, D), lambda b: (b, 0)),
                  pl.BlockSpec((EB, DE), lambda b: (b, 0)),
                  pl.BlockSpec((EB, 16), lambda b: (b, 0)),
                  pl.BlockSpec((D, HD), lambda b: (0, 0)),
                  pl.BlockSpec((DE, HD), lambda b: (0, 0))],
        out_specs=pl.BlockSpec((EB, D), lambda b: (b, 0)),
        out_shape=jax.ShapeDtypeStruct((E, D), jnp.float32),
    )(A_j, edge_fea, alphaf, Wx_bf, We)


def _outuv_body(p_ref, bias_ref, w1a_ref, w1b_ref, b1_ref,
                out_ref, u_ref, v_ref):
    s = p_ref[0] + p_ref[1] + bias_ref[...]
    out_ref[...] = s
    u_ref[...] = jnp.dot(s, w1a_ref[...],
                         preferred_element_type=jnp.float32) + b1_ref[...]
    v_ref[...] = jnp.dot(s, w1b_ref[...], preferred_element_type=jnp.float32)


def _tc_outuv(P, bias2d, W1a, W1b, b1_2d):
    return pl.pallas_call(
        _outuv_body,
        grid=(NNB,),
        in_specs=[pl.BlockSpec((NC, NB, D), lambda b: (0, b, 0)),
                  pl.BlockSpec((1, D), lambda b: (0, 0)),
                  pl.BlockSpec((D, D), lambda b: (0, 0)),
                  pl.BlockSpec((D, D), lambda b: (0, 0)),
                  pl.BlockSpec((1, D), lambda b: (0, 0))],
        out_specs=[pl.BlockSpec((NB, D), lambda b: (b, 0)),
                   pl.BlockSpec((NB, D), lambda b: (b, 0)),
                   pl.BlockSpec((NB, D), lambda b: (b, 0))],
        out_shape=[jax.ShapeDtypeStruct((N, D), jnp.float32)] * 3,
    )(P, bias2d, W1a, W1b, b1_2d)


def _edge_body(ur_ref, vc_ref, ef_ref, w1c_ref, w2_ref, b2_ref, o_ref):
    pre = (ur_ref[...] + vc_ref[...]
           + jnp.dot(ef_ref[...], w1c_ref[...],
                     preferred_element_type=jnp.float32))
    hh = _silu(pre)
    t = jnp.dot(hh, w2_ref[...], preferred_element_type=jnp.float32) + b2_ref[...]
    o_ref[...] = _silu(t)


def _tc_edge(Ur, Vc, edge_fea, W1c, W2, b2_2d):
    return pl.pallas_call(
        _edge_body,
        grid=(NEB,),
        in_specs=[pl.BlockSpec((EB, D), lambda b: (b, 0)),
                  pl.BlockSpec((EB, D), lambda b: (b, 0)),
                  pl.BlockSpec((EB, DE), lambda b: (b, 0)),
                  pl.BlockSpec((DE, D), lambda b: (0, 0)),
                  pl.BlockSpec((D, DE), lambda b: (0, 0)),
                  pl.BlockSpec((1, DE), lambda b: (0, 0))],
        out_specs=pl.BlockSpec((EB, DE), lambda b: (b, 0)),
        out_shape=jax.ShapeDtypeStruct((E, DE), jnp.float32),
    )(Ur, Vc, edge_fea, W1c, W2, b2_2d)


def _make_gather2(width, dtype=jnp.float32):
    @functools.partial(
        pl.kernel,
        out_type=[jax.ShapeDtypeStruct((E, width), dtype)] * 2,
        mesh=_mesh,
        scratch_types=[
            pltpu.VMEM((CHUNK,), jnp.int32),
            pltpu.VMEM((CHUNK, width), dtype),
            pltpu.SemaphoreType.DMA,
        ],
    )
    def k(tab_a, idx_a, tab_b, idx_b, out_a, out_b, idx_v, rows_v, sem):
        wid = lax.axis_index("s") * NC + lax.axis_index("c")
        base0 = wid * PER_W

        def body(ci, _):
            base = base0 + ci * CHUNK
            pltpu.sync_copy(idx_a.at[pl.ds(base, CHUNK)], idx_v)
            pltpu.async_copy(tab_a.at[idx_v], rows_v, sem).wait()
            pltpu.sync_copy(rows_v, out_a.at[pl.ds(base, CHUNK)])
            pltpu.sync_copy(idx_b.at[pl.ds(base, CHUNK)], idx_v)
            pltpu.async_copy(tab_b.at[idx_v], rows_v, sem).wait()
            pltpu.sync_copy(rows_v, out_b.at[pl.ds(base, CHUNK)])
            return _

        lax.fori_loop(0, NCH_W, body, 0)

    return k


@functools.partial(
    pl.kernel,
    out_type=jax.ShapeDtypeStruct((E, 16), jnp.float32),
    mesh=_mesh,
    scratch_types=[
        pltpu.VMEM((CHUNK,), jnp.int32),
        pltpu.VMEM((CHUNK, 16), jnp.float32),
        pltpu.VMEM((CHUNK, D), jnp.float32),
        pltpu.VMEM((CHUNK, D), jnp.float32),
        pltpu.VMEM_SHARED((N, D), jnp.float32),
    ],
)
def _sc_softmax_den(alpha16, idx_i, zeros128, out, idx_v, av, ev, dv, den_sp):
    c = lax.axis_index("c")
    t = lax.axis_index("s")

    @pl.when(t < N // ROWS_T)
    def _():
        pltpu.sync_copy(zeros128, den_sp.at[pl.ds(t * ROWS_T, ROWS_T)])

    pltpu.sync_copy(zeros128.at[pl.ds(0, CHUNK)], ev)
    plsc.subcore_barrier()

    def body_a(ci, _):
        base = t * PER_T + ci * CHUNK
        pltpu.sync_copy(idx_i.at[pl.ds(base, CHUNK)], idx_v)
        pltpu.sync_copy(alpha16.at[pl.ds(base, CHUNK)], av)

        def expo(r, __):
            ev[r, pl.ds(0, 16)] = jnp.exp(av[r])
            return __

        lax.fori_loop(0, CHUNK, expo, 0)
        pltpu.sync_copy(ev, den_sp.at[idx_v], add=True)
        return _

    lax.fori_loop(0, NCH_T, body_a, 0)
    plsc.subcore_barrier()

    wid = t * NC + c
    base0 = wid * PER_W

    def body_b(ci, _):
        base = base0 + ci * CHUNK
        pltpu.sync_copy(idx_i.at[pl.ds(base, CHUNK)], idx_v)
        pltpu.sync_copy(alpha16.at[pl.ds(base, CHUNK)], av)
        pltpu.sync_copy(den_sp.at[idx_v], dv)

        def norm(r, __):
            av[r] = jnp.exp(av[r]) / (dv[r, pl.ds(0, 16)] + 1e-16)
            return __

        lax.fori_loop(0, CHUNK, norm, 0)
        pltpu.sync_copy(av, out.at[pl.ds(base, CHUNK)])
        return _

    lax.fori_loop(0, NCH_W, body_b, 0)


@functools.partial(
    pl.kernel,
    out_type=jax.ShapeDtypeStruct((NC, N, D), jnp.float32),
    mesh=_mesh,
    scratch_types=[
        pltpu.VMEM((CHUNK,), jnp.int32),
        pltpu.VMEM((CHUNK, D), jnp.float32),
        pltpu.VMEM_SHARED((N, D), jnp.float32),
    ],
)
def _sc_aggr(msum, idx_i, zeros128, out, idx_v, rows_v, acc_sp):
    c = lax.axis_index("c")
    t = lax.axis_index("s")

    @pl.when(t < N // ROWS_T)
    def _():
        pltpu.sync_copy(zeros128, acc_sp.at[pl.ds(t * ROWS_T, ROWS_T)])

    plsc.subcore_barrier()

    base0 = c * (E // NC) + t * PER_W

    def body(ci, _):
        base = base0 + ci * CHUNK
        pltpu.sync_copy(idx_i.at[pl.ds(base, CHUNK)], idx_v)
        pltpu.sync_copy(msum.at[pl.ds(base, CHUNK)], rows_v)
        pltpu.sync_copy(rows_v, acc_sp.at[idx_v], add=True)
        return _

    lax.fori_loop(0, NCH_W, body, 0)
    plsc.subcore_barrier()

    @pl.when(t < N // ROWS_T)
    def _():
        pltpu.sync_copy(acc_sp.at[pl.ds(t * ROWS_T, ROWS_T)],
                        out.at[c, pl.ds(t * ROWS_T, ROWS_T)])


_gather2_128 = _make_gather2(D)


def kernel(atom_fea, edge_idx, edge_fea, batch, distance, edge_vec,
           W, att, bias, bn_gamma, bn_beta, W1, b1, W2, b2):
    i = edge_idx[0]
    j = edge_idx[1]
    Wx = W[:D]
    We = W[D:]
    att1 = att[0, :, :D]
    att2 = att[0, :, D:]
    g2 = jnp.zeros((1, 16), jnp.float32).at[0, :H].set(
        bn_gamma / jnp.sqrt(1.0 + 1e-5))
    beta = jnp.zeros((1, 16), jnp.float32).at[0, :H].set(bn_beta)
    zeros128 = jnp.zeros((ROWS_T, D), jnp.float32)
    del batch, distance, edge_vec

    Wx_bf = Wx.astype(jnp.bfloat16)
    A_i, A_j = _gather2_128(atom_fea, i, atom_fea, j)
    alpha16 = _tc_alpha(A_i, A_j, edge_fea, Wx_bf, We, att1, att2, g2, beta)
    alphaf = _sc_softmax_den(alpha16, i, zeros128)
    msum = _tc_msum(A_j, edge_fea, alphaf, Wx_bf, We)
    P = _sc_aggr(msum, i, zeros128)
    out, U, V = _tc_outuv(P, bias[None], W1[:D], W1[D:2 * D], b1[None])
    Ur, Vc = _gather2_128(U, i, V, j)
    e = _tc_edge(Ur, Vc, edge_fea, W1[2 * D:], W2, b2[None])
    return (out, e)


# speedup vs baseline: 1.1149x; 1.1149x over previous
import functools

import jax
import jax.numpy as jnp
from jax import lax
from jax.experimental import pallas as pl
from jax.experimental.pallas import tpu as pltpu
from jax.experimental.pallas import tpu_sc as plsc

N, E, D, DE, H = 10000, 320000, 128, 16, 4
HD = H * D
NC, NS, LL = 2, 16, 16
NW = NC * NS
CHUNK = 80
PER_W = E // NW
NCH_W = PER_W // CHUNK
PER_T = E // NS
NCH_T = PER_T // CHUNK
ROWS_T = 1000
EB = 512
NEB = E // EB
NB = 2000
NNB = N // NB

_mesh = plsc.VectorSubcoreMesh(core_axis_name="c", subcore_axis_name="s",
                               num_cores=NC, num_subcores=NS)


def _sp(x):
    return jnp.maximum(x, 0.0) + jnp.log1p(jnp.exp(-jnp.abs(x)))


def _silu(x):
    return x * (1.0 / (1.0 + jnp.exp(-x)))


def _alpha_body(ai_ref, aj_ref, ef_ref, wx_ref, we_ref,
                ma_ref, mb_ref, g2_ref, beta_ref, o_ref, xj_ref):
    et = jnp.dot(ef_ref[...], we_ref[...], preferred_element_type=jnp.float32)
    xi = _sp(jnp.dot(ai_ref[...].astype(jnp.bfloat16), wx_ref[...],
                     preferred_element_type=jnp.float32) + et)
    xj = _sp(jnp.dot(aj_ref[...].astype(jnp.bfloat16), wx_ref[...],
                     preferred_element_type=jnp.float32) + et)
    xj_ref[...] = xj.astype(jnp.bfloat16)
    draw = (jnp.dot(xi, ma_ref[...], preferred_element_type=jnp.float32)
            + jnp.dot(xj, mb_ref[...], preferred_element_type=jnp.float32))
    draw16 = jnp.concatenate([draw, jnp.zeros((EB, 16 - H), jnp.float32)],
                             axis=1)
    o_ref[...] = _sp(_sp(draw16) * g2_ref[...] + beta_ref[...])


def _tc_alpha(A_i, A_j, edge_fea, Wx_bf, We, Ma, Mb, g2, beta):
    return pl.pallas_call(
        _alpha_body,
        grid=(NEB,),
        in_specs=[pl.BlockSpec((EB, D), lambda b: (b, 0)),
                  pl.BlockSpec((EB, D), lambda b: (b, 0)),
                  pl.BlockSpec((EB, DE), lambda b: (b, 0)),
                  pl.BlockSpec((D, HD), lambda b: (0, 0)),
                  pl.BlockSpec((DE, HD), lambda b: (0, 0)),
                  pl.BlockSpec((HD, H), lambda b: (0, 0)),
                  pl.BlockSpec((HD, H), lambda b: (0, 0)),
                  pl.BlockSpec((1, 16), lambda b: (0, 0)),
                  pl.BlockSpec((1, 16), lambda b: (0, 0))],
        out_specs=[pl.BlockSpec((EB, 16), lambda b: (b, 0)),
                   pl.BlockSpec((EB, HD), lambda b: (b, 0))],
        out_shape=[jax.ShapeDtypeStruct((E, 16), jnp.float32),
                   jax.ShapeDtypeStruct((E, HD), jnp.bfloat16)],
    )(A_i, A_j, edge_fea, Wx_bf, We, Ma, Mb, g2, beta)


def _msum_body(xj_ref, af_ref, o_ref):
    xj = xj_ref[...].astype(jnp.float32)
    af = af_ref[...]
    acc = xj[:, 0:D] * af[:, 0:1]
    for h in range(1, H):
        acc = acc + xj[:, h * D:(h + 1) * D] * af[:, h:h + 1]
    o_ref[...] = acc * (1.0 / H)


def _tc_msum(xjb, alphaf):
    return pl.pallas_call(
        _msum_body,
        grid=(NEB,),
        in_specs=[pl.BlockSpec((EB, HD), lambda b: (b, 0)),
                  pl.BlockSpec((EB, 16), lambda b: (b, 0))],
        out_specs=pl.BlockSpec((EB, D), lambda b: (b, 0)),
        out_shape=jax.ShapeDtypeStruct((E, D), jnp.float32),
    )(xjb, alphaf)


def _outuv_body(p_ref, bias_ref, w1a_ref, w1b_ref, b1_ref,
                out_ref, u_ref, v_ref):
    s = p_ref[0] + p_ref[1] + bias_ref[...]
    out_ref[...] = s
    u_ref[...] = jnp.dot(s, w1a_ref[...],
                         preferred_element_type=jnp.float32) + b1_ref[...]
    v_ref[...] = jnp.dot(s, w1b_ref[...], preferred_element_type=jnp.float32)


def _tc_outuv(P, bias2d, W1a, W1b, b1_2d):
    return pl.pallas_call(
        _outuv_body,
        grid=(NNB,),
        in_specs=[pl.BlockSpec((NC, NB, D), lambda b: (0, b, 0)),
                  pl.BlockSpec((1, D), lambda b: (0, 0)),
                  pl.BlockSpec((D, D), lambda b: (0, 0)),
                  pl.BlockSpec((D, D), lambda b: (0, 0)),
                  pl.BlockSpec((1, D), lambda b: (0, 0))],
        out_specs=[pl.BlockSpec((NB, D), lambda b: (b, 0)),
                   pl.BlockSpec((NB, D), lambda b: (b, 0)),
                   pl.BlockSpec((NB, D), lambda b: (b, 0))],
        out_shape=[jax.ShapeDtypeStruct((N, D), jnp.float32)] * 3,
    )(P, bias2d, W1a, W1b, b1_2d)


def _edge_body(ur_ref, vc_ref, ef_ref, w1c_ref, w2_ref, b2_ref, o_ref):
    pre = (ur_ref[...] + vc_ref[...]
           + jnp.dot(ef_ref[...], w1c_ref[...],
                     preferred_element_type=jnp.float32))
    hh = _silu(pre)
    t = jnp.dot(hh, w2_ref[...], preferred_element_type=jnp.float32) + b2_ref[...]
    o_ref[...] = _silu(t)


def _tc_edge(Ur, Vc, edge_fea, W1c, W2, b2_2d):
    return pl.pallas_call(
        _edge_body,
        grid=(NEB,),
        in_specs=[pl.BlockSpec((EB, D), lambda b: (b, 0)),
                  pl.BlockSpec((EB, D), lambda b: (b, 0)),
                  pl.BlockSpec((EB, DE), lambda b: (b, 0)),
                  pl.BlockSpec((DE, D), lambda b: (0, 0)),
                  pl.BlockSpec((D, DE), lambda b: (0, 0)),
                  pl.BlockSpec((1, DE), lambda b: (0, 0))],
        out_specs=pl.BlockSpec((EB, DE), lambda b: (b, 0)),
        out_shape=jax.ShapeDtypeStruct((E, DE), jnp.float32),
    )(Ur, Vc, edge_fea, W1c, W2, b2_2d)


def _make_gather2(width, dtype=jnp.float32):
    nwin = PER_W // 5

    @functools.partial(
        pl.kernel,
        out_type=[jax.ShapeDtypeStruct((E, width), dtype)] * 2,
        mesh=_mesh,
        scratch_types=[
            pltpu.VMEM((nwin,), jnp.int32),
            pltpu.VMEM((CHUNK, width), dtype),
            pltpu.SemaphoreType.DMA,
        ],
    )
    def k(tab_a, idx_a, tab_b, idx_b, out_a, out_b, idx_v, rows_v, sem):
        wid = lax.axis_index("s") * NC + lax.axis_index("c")
        base0 = wid * PER_W
        cpw = nwin // CHUNK

        for tab, idx, out in ((tab_a, idx_a, out_a), (tab_b, idx_b, out_b)):
            def wloop(w, _, tab=tab, idx=idx, out=out):
                pltpu.sync_copy(idx.at[pl.ds(base0 + w * nwin, nwin)], idx_v)

                def body(ci, __, tab=tab, out=out):
                    base = base0 + w * nwin + ci * CHUNK
                    pltpu.async_copy(
                        tab.at[idx_v.at[pl.ds(ci * CHUNK, CHUNK)]], rows_v,
                        sem).wait()
                    pltpu.sync_copy(rows_v, out.at[pl.ds(base, CHUNK)])
                    return __

                lax.fori_loop(0, cpw, body, 0)
                return _

            lax.fori_loop(0, PER_W // nwin, wloop, 0)

    return k


SUPG = 5
SUPR = SUPG * CHUNK


@functools.partial(
    pl.kernel,
    out_type=jax.ShapeDtypeStruct((E * 16,), jnp.float32),
    mesh=_mesh,
    scratch_types=[
        pltpu.VMEM((CHUNK,), jnp.int32),
        pltpu.VMEM((SUPR * 16,), jnp.float32),
        pltpu.VMEM((CHUNK, D), jnp.float32),
        pltpu.VMEM_SHARED((N, D), jnp.float32),
    ],
)
def _sc_softmax_den(alpha_fl, idx_i, zeros128, out, idxv, av, ev, den_sp):
    c = lax.axis_index("c")
    t = lax.axis_index("s")

    @pl.when(t < N // ROWS_T)
    def _():
        def zz(z, _):
            pltpu.sync_copy(zeros128.at[pl.ds(z * 200, 200)],
                            den_sp.at[pl.ds(t * ROWS_T + z * 200, 200)])
            return _

        lax.fori_loop(0, 5, zz, 0)

    pltpu.sync_copy(zeros128.at[pl.ds(0, CHUNK)], ev)
    plsc.subcore_barrier()

    def body_a(s, _):
        pltpu.sync_copy(
            alpha_fl.at[pl.ds((t * PER_T + s * SUPR) * 16, SUPR * 16)], av)

        def grp(g, __):
            def rowf(r, ___):
                ev[r, pl.ds(0, 16)] = jnp.exp(
                    av[pl.ds((g * CHUNK + r) * 16, 16)])
                return ___

            lax.fori_loop(0, CHUNK, rowf, 0)
            pltpu.sync_copy(
                idx_i.at[pl.ds(t * PER_T + (s * SUPG + g) * CHUNK, CHUNK)],
                idxv)
            pltpu.sync_copy(ev, den_sp.at[idxv], add=True)
            return __

        lax.fori_loop(0, SUPG, grp, 0)
        return _

    lax.fori_loop(0, NCH_T // SUPG, body_a, 0)
    plsc.subcore_barrier()

    wid = t * NC + c

    def body_b(s, _):
        base = wid * PER_W + s * SUPR
        pltpu.sync_copy(alpha_fl.at[pl.ds(base * 16, SUPR * 16)], av)

        def grp(g, __):
            pltpu.sync_copy(
                idx_i.at[pl.ds(wid * PER_W + (s * SUPG + g) * CHUNK, CHUNK)],
                idxv)
            pltpu.sync_copy(den_sp.at[idxv], ev)

            def rowf(r, ___):
                rr = pl.ds((g * CHUNK + r) * 16, 16)
                av[rr] = jnp.exp(av[rr]) / (ev[r, pl.ds(0, 16)] + 1e-16)
                return ___

            lax.fori_loop(0, CHUNK, rowf, 0)
            return __

        lax.fori_loop(0, SUPG, grp, 0)
        pltpu.sync_copy(av, out.at[pl.ds(base * 16, SUPR * 16)])
        return _

    lax.fori_loop(0, NCH_W // SUPG, body_b, 0)


@functools.partial(
    pl.kernel,
    out_type=jax.ShapeDtypeStruct((NC, N, D), jnp.float32),
    mesh=_mesh,
    scratch_types=[
        pltpu.VMEM((CHUNK,), jnp.int32),
        pltpu.VMEM((CHUNK, D), jnp.float32),
        pltpu.VMEM_SHARED((N, D), jnp.float32),
    ],
)
def _sc_aggr(msum, idx_i, zeros128, out, idxv, rows_v, acc_sp):
    c = lax.axis_index("c")
    t = lax.axis_index("s")

    @pl.when(t < N // ROWS_T)
    def _():
        pltpu.sync_copy(zeros128, acc_sp.at[pl.ds(t * ROWS_T, ROWS_T)])

    wid = t * NC + c
    plsc.subcore_barrier()

    base0 = wid * PER_W

    def body(ci, _):
        pltpu.sync_copy(idx_i.at[pl.ds(base0 + ci * CHUNK, CHUNK)], idxv)
        pltpu.sync_copy(msum.at[pl.ds(base0 + ci * CHUNK, CHUNK)], rows_v)
        pltpu.sync_copy(rows_v, acc_sp.at[idxv], add=True)
        return _

    lax.fori_loop(0, NCH_W, body, 0)
    plsc.subcore_barrier()

    @pl.when(t < N // ROWS_T)
    def _():
        pltpu.sync_copy(acc_sp.at[pl.ds(t * ROWS_T, ROWS_T)],
                        out.at[c, pl.ds(t * ROWS_T, ROWS_T)])


_gather2_128 = _make_gather2(D)


def kernel(atom_fea, edge_idx, edge_fea, batch, distance, edge_vec,
           W, att, bias, bn_gamma, bn_beta, W1, b1, W2, b2):
    i = edge_idx[0]
    j = edge_idx[1]
    Wx = W[:D]
    We = W[D:]
    att1 = att[0, :, :D]
    att2 = att[0, :, D:]
    dd = jnp.arange(HD)
    Ma = jnp.zeros((HD, H), jnp.float32).at[dd, dd // D].set(att1.reshape(-1))
    Mb = jnp.zeros((HD, H), jnp.float32).at[dd, dd // D].set(att2.reshape(-1))
    g2 = jnp.zeros((1, 16), jnp.float32).at[0, :H].set(
        bn_gamma / jnp.sqrt(1.0 + 1e-5))
    beta = jnp.zeros((1, 16), jnp.float32).at[0, :H].set(bn_beta)
    zeros128 = jnp.zeros((ROWS_T, D), jnp.float32)
    del batch, distance, edge_vec

    Wx_bf = Wx.astype(jnp.bfloat16)
    A_i, A_j = _gather2_128(atom_fea, i, atom_fea, j)
    alpha16, xjb = _tc_alpha(A_i, A_j, edge_fea, Wx_bf, We, Ma, Mb, g2, beta)
    alphaf = _sc_softmax_den(alpha16.reshape(-1), i, zeros128)
    msum = _tc_msum(xjb, alphaf.reshape(E, 16))
    P = _sc_aggr(msum, i, zeros128)
    out, U, V = _tc_outuv(P, bias[None], W1[:D], W1[D:2 * D], b1[None])
    Ur, Vc = _gather2_128(U, i, V, j)
    e = _tc_edge(Ur, Vc, edge_fea, W1[2 * D:], W2, b2[None])
    return (out, e)


# speedup vs baseline: 1.1214x; 1.0058x over previous
import functools

import jax
import jax.numpy as jnp
from jax import lax
from jax.experimental import pallas as pl
from jax.experimental.pallas import tpu as pltpu
from jax.experimental.pallas import tpu_sc as plsc

N, E, D, DE, H = 10000, 320000, 128, 16, 4
HD = H * D
NC, NS, LL = 2, 16, 16
NW = NC * NS
CHUNK = 80
PER_W = E // NW
NCH_W = PER_W // CHUNK
PER_T = E // NS
NCH_T = PER_T // CHUNK
ROWS_T = 1000
EB = 512
NEB = E // EB
NB = 2000
NNB = N // NB

_mesh = plsc.VectorSubcoreMesh(core_axis_name="c", subcore_axis_name="s",
                               num_cores=NC, num_subcores=NS)


def _sp(x):
    return jnp.maximum(x, 0.0) + jnp.log1p(jnp.exp(-jnp.abs(x)))


def _silu(x):
    return x * (1.0 / (1.0 + jnp.exp(-x)))


def _alpha_body(ai_ref, aj_ref, ef_ref, wx_ref, we_ref,
                ma_ref, mb_ref, g2_ref, beta_ref, o_ref, xj_ref):
    et = jnp.dot(ef_ref[...], we_ref[...], preferred_element_type=jnp.float32)
    xi = _sp(jnp.dot(ai_ref[...].astype(jnp.bfloat16), wx_ref[...],
                     preferred_element_type=jnp.float32) + et)
    xj = _sp(jnp.dot(aj_ref[...].astype(jnp.bfloat16), wx_ref[...],
                     preferred_element_type=jnp.float32) + et)
    xj_ref[...] = xj.astype(jnp.bfloat16)
    draw = (jnp.dot(xi, ma_ref[...], preferred_element_type=jnp.float32)
            + jnp.dot(xj, mb_ref[...], preferred_element_type=jnp.float32))
    draw16 = jnp.concatenate([draw, jnp.zeros((EB, 16 - H), jnp.float32)],
                             axis=1)
    o_ref[...] = _sp(_sp(draw16) * g2_ref[...] + beta_ref[...])


def _tc_alpha(A_i, A_j, edge_fea, Wx_bf, We, Ma, Mb, g2, beta):
    return pl.pallas_call(
        _alpha_body,
        grid=(NEB,),
        in_specs=[pl.BlockSpec((EB, D), lambda b: (b, 0)),
                  pl.BlockSpec((EB, D), lambda b: (b, 0)),
                  pl.BlockSpec((EB, DE), lambda b: (b, 0)),
                  pl.BlockSpec((D, HD), lambda b: (0, 0)),
                  pl.BlockSpec((DE, HD), lambda b: (0, 0)),
                  pl.BlockSpec((HD, H), lambda b: (0, 0)),
                  pl.BlockSpec((HD, H), lambda b: (0, 0)),
                  pl.BlockSpec((1, 16), lambda b: (0, 0)),
                  pl.BlockSpec((1, 16), lambda b: (0, 0))],
        out_specs=[pl.BlockSpec((EB, 16), lambda b: (b, 0)),
                   pl.BlockSpec((EB, HD), lambda b: (b, 0))],
        out_shape=[jax.ShapeDtypeStruct((E, 16), jnp.float32),
                   jax.ShapeDtypeStruct((E, HD), jnp.bfloat16)],
    )(A_i, A_j, edge_fea, Wx_bf, We, Ma, Mb, g2, beta)


def _msum_body(xj_ref, af_ref, o_ref):
    xj = xj_ref[...].astype(jnp.float32)
    af = af_ref[...]
    acc = xj[:, 0:D] * af[:, 0:1]
    for h in range(1, H):
        acc = acc + xj[:, h * D:(h + 1) * D] * af[:, h:h + 1]
    o_ref[...] = acc * (1.0 / H)


def _tc_msum(xjb, alphaf):
    return pl.pallas_call(
        _msum_body,
        grid=(NEB,),
        in_specs=[pl.BlockSpec((EB, HD), lambda b: (b, 0)),
                  pl.BlockSpec((EB, 16), lambda b: (b, 0))],
        out_specs=pl.BlockSpec((EB, D), lambda b: (b, 0)),
        out_shape=jax.ShapeDtypeStruct((E, D), jnp.float32),
    )(xjb, alphaf)


def _outuv_body(p_ref, bias_ref, w1a_ref, w1b_ref, b1_ref,
                out_ref, u_ref, v_ref):
    s = p_ref[0] + p_ref[1] + bias_ref[...]
    out_ref[...] = s
    u_ref[...] = jnp.dot(s, w1a_ref[...],
                         preferred_element_type=jnp.float32) + b1_ref[...]
    v_ref[...] = jnp.dot(s, w1b_ref[...], preferred_element_type=jnp.float32)


def _tc_outuv(P, bias2d, W1a, W1b, b1_2d):
    return pl.pallas_call(
        _outuv_body,
        grid=(NNB,),
        in_specs=[pl.BlockSpec((NC, NB, D), lambda b: (0, b, 0)),
                  pl.BlockSpec((1, D), lambda b: (0, 0)),
                  pl.BlockSpec((D, D), lambda b: (0, 0)),
                  pl.BlockSpec((D, D), lambda b: (0, 0)),
                  pl.BlockSpec((1, D), lambda b: (0, 0))],
        out_specs=[pl.BlockSpec((NB, D), lambda b: (b, 0)),
                   pl.BlockSpec((NB, D), lambda b: (b, 0)),
                   pl.BlockSpec((NB, D), lambda b: (b, 0))],
        out_shape=[jax.ShapeDtypeStruct((N, D), jnp.float32)] * 3,
    )(P, bias2d, W1a, W1b, b1_2d)


def _edge_body(ur_ref, vc_ref, ef_ref, w1c_ref, w2_ref, b2_ref, o_ref):
    pre = (ur_ref[...] + vc_ref[...]
           + jnp.dot(ef_ref[...], w1c_ref[...],
                     preferred_element_type=jnp.float32))
    hh = _silu(pre)
    t = jnp.dot(hh, w2_ref[...], preferred_element_type=jnp.float32) + b2_ref[...]
    o_ref[...] = _silu(t)


def _tc_edge(Ur, Vc, edge_fea, W1c, W2, b2_2d):
    return pl.pallas_call(
        _edge_body,
        grid=(NEB,),
        in_specs=[pl.BlockSpec((EB, D), lambda b: (b, 0)),
                  pl.BlockSpec((EB, D), lambda b: (b, 0)),
                  pl.BlockSpec((EB, DE), lambda b: (b, 0)),
                  pl.BlockSpec((DE, D), lambda b: (0, 0)),
                  pl.BlockSpec((D, DE), lambda b: (0, 0)),
                  pl.BlockSpec((1, DE), lambda b: (0, 0))],
        out_specs=pl.BlockSpec((EB, DE), lambda b: (b, 0)),
        out_shape=jax.ShapeDtypeStruct((E, DE), jnp.float32),
    )(Ur, Vc, edge_fea, W1c, W2, b2_2d)


def _make_gather2(width, dtype=jnp.float32):
    nwin = PER_W // 5

    @functools.partial(
        pl.kernel,
        out_type=[jax.ShapeDtypeStruct((E, width), dtype)] * 2,
        mesh=_mesh,
        scratch_types=[
            pltpu.VMEM((nwin,), jnp.int32),
            pltpu.VMEM((CHUNK, width), dtype),
            pltpu.SemaphoreType.DMA,
        ],
    )
    def k(tab_a, idx_a, tab_b, idx_b, out_a, out_b, idx_v, rows_v, sem):
        wid = lax.axis_index("s") * NC + lax.axis_index("c")
        base0 = wid * PER_W
        cpw = nwin // CHUNK

        for tab, idx, out in ((tab_a, idx_a, out_a), (tab_b, idx_b, out_b)):
            def wloop(w, _, tab=tab, idx=idx, out=out):
                pltpu.sync_copy(idx.at[pl.ds(base0 + w * nwin, nwin)], idx_v)

                def body(ci, __, tab=tab, out=out):
                    base = base0 + w * nwin + ci * CHUNK
                    pltpu.async_copy(
                        tab.at[idx_v.at[pl.ds(ci * CHUNK, CHUNK)]], rows_v,
                        sem).wait()
                    pltpu.sync_copy(rows_v, out.at[pl.ds(base, CHUNK)])
                    return __

                lax.fori_loop(0, cpw, body, 0)
                return _

            lax.fori_loop(0, PER_W // nwin, wloop, 0)

    return k


SUPG = 5
SUPR = SUPG * CHUNK


@functools.partial(
    pl.kernel,
    out_type=jax.ShapeDtypeStruct((E * 16,), jnp.float32),
    mesh=_mesh,
    scratch_types=[
        pltpu.VMEM((CHUNK,), jnp.int32),
        pltpu.VMEM((SUPR * 16,), jnp.float32),
        pltpu.VMEM((CHUNK, D), jnp.float32),
        pltpu.VMEM_SHARED((N, D), jnp.float32),
    ],
)
def _sc_softmax_den(alpha_fl, idx_i, zeros128, out, idxv, av, ev, den_sp):
    c = lax.axis_index("c")
    t = lax.axis_index("s")

    @pl.when(t < N // ROWS_T)
    def _():
        def zz(z, _):
            pltpu.sync_copy(zeros128.at[pl.ds(z * 200, 200)],
                            den_sp.at[pl.ds(t * ROWS_T + z * 200, 200)])
            return _

        lax.fori_loop(0, 5, zz, 0)

    pltpu.sync_copy(zeros128.at[pl.ds(0, CHUNK)], ev)
    plsc.subcore_barrier()

    def body_a(s, _):
        pltpu.sync_copy(
            alpha_fl.at[pl.ds((t * PER_T + s * SUPR) * 16, SUPR * 16)], av)

        def grp(g, __):
            for r in range(CHUNK):
                ev[r, pl.ds(0, 16)] = jnp.exp(
                    av[pl.ds((g * CHUNK + r) * 16, 16)])
            pltpu.sync_copy(
                idx_i.at[pl.ds(t * PER_T + (s * SUPG + g) * CHUNK, CHUNK)],
                idxv)
            pltpu.sync_copy(ev, den_sp.at[idxv], add=True)
            return __

        lax.fori_loop(0, SUPG, grp, 0)
        return _

    lax.fori_loop(0, NCH_T // SUPG, body_a, 0)
    plsc.subcore_barrier()

    wid = t * NC + c

    def body_b(s, _):
        base = wid * PER_W + s * SUPR
        pltpu.sync_copy(alpha_fl.at[pl.ds(base * 16, SUPR * 16)], av)

        def grp(g, __):
            pltpu.sync_copy(
                idx_i.at[pl.ds(wid * PER_W + (s * SUPG + g) * CHUNK, CHUNK)],
                idxv)
            pltpu.sync_copy(den_sp.at[idxv], ev)

            for r in range(CHUNK):
                rr = pl.ds((g * CHUNK + r) * 16, 16)
                av[rr] = jnp.exp(av[rr]) / (ev[r, pl.ds(0, 16)] + 1e-16)
            return __

        lax.fori_loop(0, SUPG, grp, 0)
        pltpu.sync_copy(av, out.at[pl.ds(base * 16, SUPR * 16)])
        return _

    lax.fori_loop(0, NCH_W // SUPG, body_b, 0)


@functools.partial(
    pl.kernel,
    out_type=jax.ShapeDtypeStruct((NC, N, D), jnp.float32),
    mesh=_mesh,
    scratch_types=[
        pltpu.VMEM((CHUNK,), jnp.int32),
        pltpu.VMEM((CHUNK, D), jnp.float32),
        pltpu.VMEM_SHARED((N, D), jnp.float32),
    ],
)
def _sc_aggr(msum, idx_i, zeros128, out, idxv, rows_v, acc_sp):
    c = lax.axis_index("c")
    t = lax.axis_index("s")

    @pl.when(t < N // ROWS_T)
    def _():
        pltpu.sync_copy(zeros128, acc_sp.at[pl.ds(t * ROWS_T, ROWS_T)])

    wid = t * NC + c
    plsc.subcore_barrier()

    base0 = wid * PER_W

    def body(ci, _):
        pltpu.sync_copy(idx_i.at[pl.ds(base0 + ci * CHUNK, CHUNK)], idxv)
        pltpu.sync_copy(msum.at[pl.ds(base0 + ci * CHUNK, CHUNK)], rows_v)
        pltpu.sync_copy(rows_v, acc_sp.at[idxv], add=True)
        return _

    lax.fori_loop(0, NCH_W, body, 0)
    plsc.subcore_barrier()

    @pl.when(t < N // ROWS_T)
    def _():
        pltpu.sync_copy(acc_sp.at[pl.ds(t * ROWS_T, ROWS_T)],
                        out.at[c, pl.ds(t * ROWS_T, ROWS_T)])


_gather2_128 = _make_gather2(D)


def kernel(atom_fea, edge_idx, edge_fea, batch, distance, edge_vec,
           W, att, bias, bn_gamma, bn_beta, W1, b1, W2, b2):
    i = edge_idx[0]
    j = edge_idx[1]
    Wx = W[:D]
    We = W[D:]
    att1 = att[0, :, :D]
    att2 = att[0, :, D:]
    dd = jnp.arange(HD)
    Ma = jnp.zeros((HD, H), jnp.float32).at[dd, dd // D].set(att1.reshape(-1))
    Mb = jnp.zeros((HD, H), jnp.float32).at[dd, dd // D].set(att2.reshape(-1))
    g2 = jnp.zeros((1, 16), jnp.float32).at[0, :H].set(
        bn_gamma / jnp.sqrt(1.0 + 1e-5))
    beta = jnp.zeros((1, 16), jnp.float32).at[0, :H].set(bn_beta)
    zeros128 = jnp.zeros((ROWS_T, D), jnp.float32)
    del batch, distance, edge_vec

    Wx_bf = Wx.astype(jnp.bfloat16)
    A_i, A_j = _gather2_128(atom_fea, i, atom_fea, j)
    alpha16, xjb = _tc_alpha(A_i, A_j, edge_fea, Wx_bf, We, Ma, Mb, g2, beta)
    alphaf = _sc_softmax_den(alpha16.reshape(-1), i, zeros128)
    msum = _tc_msum(xjb, alphaf.reshape(E, 16))
    P = _sc_aggr(msum, i, zeros128)
    out, U, V = _tc_outuv(P, bias[None], W1[:D], W1[D:2 * D], b1[None])
    Ur, Vc = _gather2_128(U, i, V, j)
    e = _tc_edge(Ur, Vc, edge_fea, W1[2 * D:], W2, b2[None])
    return (out, e)


# speedup vs baseline: 1.2430x; 1.1084x over previous
import functools

import jax
import jax.numpy as jnp
from jax import lax
from jax.experimental import pallas as pl
from jax.experimental.pallas import tpu as pltpu
from jax.experimental.pallas import tpu_sc as plsc

N, E, D, DE, H = 10000, 320000, 128, 16, 4
HD = H * D
NC, NS, LL = 2, 16, 16
NW = NC * NS
CHUNK = 80
PER_W = E // NW
NCH_W = PER_W // CHUNK
PER_T = E // NS
NCH_T = PER_T // CHUNK
ROWS_T = 1000
EB = 512
NEB = E // EB
NB = 2000
NNB = N // NB

_mesh = plsc.VectorSubcoreMesh(core_axis_name="c", subcore_axis_name="s",
                               num_cores=NC, num_subcores=NS)


def _sp(x):
    return jnp.maximum(x, 0.0) + jnp.log1p(jnp.exp(-jnp.abs(x)))


def _silu(x):
    return x * (1.0 / (1.0 + jnp.exp(-x)))


def _alpha_body(ai_ref, aj_ref, ef_ref, wx_ref, we_ref,
                ma_ref, mb_ref, g2_ref, beta_ref, o_ref, xj_ref):
    et = jnp.dot(ef_ref[...], we_ref[...], preferred_element_type=jnp.float32)
    xi = _sp(jnp.dot(ai_ref[...].astype(jnp.bfloat16), wx_ref[...],
                     preferred_element_type=jnp.float32) + et)
    xj = _sp(jnp.dot(aj_ref[...].astype(jnp.bfloat16), wx_ref[...],
                     preferred_element_type=jnp.float32) + et)
    xj_ref[...] = xj.astype(jnp.bfloat16)
    draw = (jnp.dot(xi, ma_ref[...], preferred_element_type=jnp.float32)
            + jnp.dot(xj, mb_ref[...], preferred_element_type=jnp.float32))
    draw16 = jnp.concatenate([draw, jnp.zeros((EB, 16 - H), jnp.float32)],
                             axis=1)
    o_ref[...] = _sp(_sp(draw16) * g2_ref[...] + beta_ref[...])


def _tc_alpha(A_i, A_j, edge_fea, Wx_bf, We, Ma, Mb, g2, beta):
    return pl.pallas_call(
        _alpha_body,
        grid=(NEB,),
        in_specs=[pl.BlockSpec((EB, D), lambda b: (b, 0)),
                  pl.BlockSpec((EB, D), lambda b: (b, 0)),
                  pl.BlockSpec((EB, DE), lambda b: (b, 0)),
                  pl.BlockSpec((D, HD), lambda b: (0, 0)),
                  pl.BlockSpec((DE, HD), lambda b: (0, 0)),
                  pl.BlockSpec((HD, H), lambda b: (0, 0)),
                  pl.BlockSpec((HD, H), lambda b: (0, 0)),
                  pl.BlockSpec((1, 16), lambda b: (0, 0)),
                  pl.BlockSpec((1, 16), lambda b: (0, 0))],
        out_specs=[pl.BlockSpec((EB, 16), lambda b: (b, 0)),
                   pl.BlockSpec((EB, HD), lambda b: (b, 0))],
        out_shape=[jax.ShapeDtypeStruct((E, 16), jnp.float32),
                   jax.ShapeDtypeStruct((E, HD), jnp.bfloat16)],
    )(A_i, A_j, edge_fea, Wx_bf, We, Ma, Mb, g2, beta)


def _msum_body(xj_ref, af_ref, o_ref):
    xj = xj_ref[...].astype(jnp.float32)
    af = af_ref[...]
    acc = xj[:, 0:D] * af[:, 0:1]
    for h in range(1, H):
        acc = acc + xj[:, h * D:(h + 1) * D] * af[:, h:h + 1]
    o_ref[...] = acc * (1.0 / H)


def _tc_msum(xjb, alphaf):
    return pl.pallas_call(
        _msum_body,
        grid=(NEB,),
        in_specs=[pl.BlockSpec((EB, HD), lambda b: (b, 0)),
                  pl.BlockSpec((EB, 16), lambda b: (b, 0))],
        out_specs=pl.BlockSpec((EB, D), lambda b: (b, 0)),
        out_shape=jax.ShapeDtypeStruct((E, D), jnp.float32),
    )(xjb, alphaf)


def _outuv_body(p_ref, bias_ref, w1a_ref, w1b_ref, b1_ref,
                out_ref, u_ref, v_ref):
    s = p_ref[0] + p_ref[1] + bias_ref[...]
    out_ref[...] = s
    u_ref[...] = jnp.dot(s, w1a_ref[...],
                         preferred_element_type=jnp.float32) + b1_ref[...]
    v_ref[...] = jnp.dot(s, w1b_ref[...], preferred_element_type=jnp.float32)


def _tc_outuv(P, bias2d, W1a, W1b, b1_2d):
    return pl.pallas_call(
        _outuv_body,
        grid=(NNB,),
        in_specs=[pl.BlockSpec((NC, NB, D), lambda b: (0, b, 0)),
                  pl.BlockSpec((1, D), lambda b: (0, 0)),
                  pl.BlockSpec((D, D), lambda b: (0, 0)),
                  pl.BlockSpec((D, D), lambda b: (0, 0)),
                  pl.BlockSpec((1, D), lambda b: (0, 0))],
        out_specs=[pl.BlockSpec((NB, D), lambda b: (b, 0)),
                   pl.BlockSpec((NB, D), lambda b: (b, 0)),
                   pl.BlockSpec((NB, D), lambda b: (b, 0))],
        out_shape=[jax.ShapeDtypeStruct((N, D), jnp.float32)] * 3,
    )(P, bias2d, W1a, W1b, b1_2d)


def _edge_body(ur_ref, vc_ref, ef_ref, w1c_ref, w2_ref, b2_ref, o_ref):
    pre = (ur_ref[...] + vc_ref[...]
           + jnp.dot(ef_ref[...], w1c_ref[...],
                     preferred_element_type=jnp.float32))
    hh = _silu(pre)
    t = jnp.dot(hh, w2_ref[...], preferred_element_type=jnp.float32) + b2_ref[...]
    o_ref[...] = _silu(t)


def _tc_edge(Ur, Vc, edge_fea, W1c, W2, b2_2d):
    return pl.pallas_call(
        _edge_body,
        grid=(NEB,),
        in_specs=[pl.BlockSpec((EB, D), lambda b: (b, 0)),
                  pl.BlockSpec((EB, D), lambda b: (b, 0)),
                  pl.BlockSpec((EB, DE), lambda b: (b, 0)),
                  pl.BlockSpec((DE, D), lambda b: (0, 0)),
                  pl.BlockSpec((D, DE), lambda b: (0, 0)),
                  pl.BlockSpec((1, DE), lambda b: (0, 0))],
        out_specs=pl.BlockSpec((EB, DE), lambda b: (b, 0)),
        out_shape=jax.ShapeDtypeStruct((E, DE), jnp.float32),
    )(Ur, Vc, edge_fea, W1c, W2, b2_2d)


def _make_gather2(width, dtype=jnp.float32):
    nwin = PER_W // 5

    @functools.partial(
        pl.kernel,
        out_type=[jax.ShapeDtypeStruct((E, width), dtype)] * 2,
        mesh=_mesh,
        scratch_types=[
            pltpu.VMEM((nwin,), jnp.int32),
            pltpu.VMEM((CHUNK, width), dtype),
            pltpu.SemaphoreType.DMA,
        ],
    )
    def k(tab_a, idx_a, tab_b, idx_b, out_a, out_b, idx_v, rows_v, sem):
        wid = lax.axis_index("s") * NC + lax.axis_index("c")
        base0 = wid * PER_W
        cpw = nwin // CHUNK

        for tab, idx, out in ((tab_a, idx_a, out_a), (tab_b, idx_b, out_b)):
            def wloop(w, _, tab=tab, idx=idx, out=out):
                pltpu.sync_copy(idx.at[pl.ds(base0 + w * nwin, nwin)], idx_v)

                def body(ci, __, tab=tab, out=out):
                    base = base0 + w * nwin + ci * CHUNK
                    pltpu.async_copy(
                        tab.at[idx_v.at[pl.ds(ci * CHUNK, CHUNK)]], rows_v,
                        sem).wait()
                    pltpu.sync_copy(rows_v, out.at[pl.ds(base, CHUNK)])
                    return __

                lax.fori_loop(0, cpw, body, 0)
                return _

            lax.fori_loop(0, PER_W // nwin, wloop, 0)

    return k


SUPG = 5
SUPR = SUPG * CHUNK


@functools.partial(
    pl.kernel,
    out_type=jax.ShapeDtypeStruct((E * 16,), jnp.float32),
    mesh=_mesh,
    scratch_types=[
        pltpu.VMEM((SUPG, CHUNK), jnp.int32),
        pltpu.VMEM((SUPR * 16,), jnp.float32),
        pltpu.VMEM((CHUNK, D), jnp.float32),
        pltpu.VMEM((CHUNK, D), jnp.float32),
        pltpu.SemaphoreType.DMA,
        pltpu.SemaphoreType.DMA,
        pltpu.VMEM_SHARED((N, D), jnp.float32),
    ],
)
def _sc_softmax_den(alpha_fl, idx4a, idx4b, zeros128, out,
                    idxv, av, ev0, ev1, sem0, sem1, den_sp):
    c = lax.axis_index("c")
    t = lax.axis_index("s")

    @pl.when(t < N // ROWS_T)
    def _():
        def zz(z, _):
            pltpu.sync_copy(zeros128.at[pl.ds(z * 200, 200)],
                            den_sp.at[pl.ds(t * ROWS_T + z * 200, 200)])
            return _

        lax.fori_loop(0, 5, zz, 0)

    pltpu.sync_copy(zeros128.at[pl.ds(0, CHUNK)], ev0)
    pltpu.sync_copy(zeros128.at[pl.ds(0, CHUNK)], ev1)
    plsc.subcore_barrier()
    evs = (ev0, ev1)
    sems = (sem0, sem1)

    def body_a(s, _):
        pltpu.sync_copy(
            alpha_fl.at[pl.ds((t * PER_T + s * SUPR) * 16, SUPR * 16)], av)
        pltpu.sync_copy(idx4a.at[t, s], idxv)
        descs = [None, None]
        for g in range(SUPG):
            b = g & 1
            if descs[b] is not None:
                descs[b].wait()
            ev = evs[b]
            for r in range(CHUNK):
                ev[r, pl.ds(0, 16)] = jnp.exp(
                    av[pl.ds((g * CHUNK + r) * 16, 16)])
            descs[b] = pltpu.async_copy(ev, den_sp.at[idxv.at[g]], sems[b],
                                        add=True)
        descs[0].wait()
        descs[1].wait()
        return _

    lax.fori_loop(0, NCH_T // SUPG, body_a, 0)
    plsc.subcore_barrier()

    wid = t * NC + c

    def body_b(s, _):
        base = wid * PER_W + s * SUPR
        pltpu.sync_copy(alpha_fl.at[pl.ds(base * 16, SUPR * 16)], av)

        pltpu.sync_copy(idx4b.at[wid, s], idxv)
        descs = [None, None]
        descs[0] = pltpu.async_copy(den_sp.at[idxv.at[0]], ev0, sem0)
        for g in range(SUPG):
            b = g & 1
            descs[b].wait()
            if g + 1 < SUPG:
                nb = (g + 1) & 1
                descs[nb] = pltpu.async_copy(den_sp.at[idxv.at[g + 1]],
                                             evs[nb], sems[nb])
            ev = evs[b]
            for r in range(CHUNK):
                rr = pl.ds((g * CHUNK + r) * 16, 16)
                av[rr] = jnp.exp(av[rr]) / (ev[r, pl.ds(0, 16)] + 1e-16)
        pltpu.sync_copy(av, out.at[pl.ds(base * 16, SUPR * 16)])
        return _

    lax.fori_loop(0, NCH_W // SUPG, body_b, 0)


@functools.partial(
    pl.kernel,
    out_type=jax.ShapeDtypeStruct((NC, N, D), jnp.float32),
    mesh=_mesh,
    scratch_types=[
        pltpu.VMEM((CHUNK,), jnp.int32),
        pltpu.VMEM((CHUNK, D), jnp.float32),
        pltpu.VMEM_SHARED((N, D), jnp.float32),
    ],
)
def _sc_aggr(msum, idx_i, zeros128, out, idxv, rows_v, acc_sp):
    c = lax.axis_index("c")
    t = lax.axis_index("s")

    @pl.when(t < N // ROWS_T)
    def _():
        pltpu.sync_copy(zeros128, acc_sp.at[pl.ds(t * ROWS_T, ROWS_T)])

    wid = t * NC + c
    plsc.subcore_barrier()

    base0 = wid * PER_W

    def body(ci, _):
        pltpu.sync_copy(idx_i.at[pl.ds(base0 + ci * CHUNK, CHUNK)], idxv)
        pltpu.sync_copy(msum.at[pl.ds(base0 + ci * CHUNK, CHUNK)], rows_v)
        pltpu.sync_copy(rows_v, acc_sp.at[idxv], add=True)
        return _

    lax.fori_loop(0, NCH_W, body, 0)
    plsc.subcore_barrier()

    @pl.when(t < N // ROWS_T)
    def _():
        pltpu.sync_copy(acc_sp.at[pl.ds(t * ROWS_T, ROWS_T)],
                        out.at[c, pl.ds(t * ROWS_T, ROWS_T)])


_gather2_128 = _make_gather2(D)


def kernel(atom_fea, edge_idx, edge_fea, batch, distance, edge_vec,
           W, att, bias, bn_gamma, bn_beta, W1, b1, W2, b2):
    i = edge_idx[0]
    j = edge_idx[1]
    Wx = W[:D]
    We = W[D:]
    att1 = att[0, :, :D]
    att2 = att[0, :, D:]
    dd = jnp.arange(HD)
    Ma = jnp.zeros((HD, H), jnp.float32).at[dd, dd // D].set(att1.reshape(-1))
    Mb = jnp.zeros((HD, H), jnp.float32).at[dd, dd // D].set(att2.reshape(-1))
    g2 = jnp.zeros((1, 16), jnp.float32).at[0, :H].set(
        bn_gamma / jnp.sqrt(1.0 + 1e-5))
    beta = jnp.zeros((1, 16), jnp.float32).at[0, :H].set(bn_beta)
    zeros128 = jnp.zeros((ROWS_T, D), jnp.float32)
    del batch, distance, edge_vec

    Wx_bf = Wx.astype(jnp.bfloat16)
    A_i, A_j = _gather2_128(atom_fea, i, atom_fea, j)
    alpha16, xjb = _tc_alpha(A_i, A_j, edge_fea, Wx_bf, We, Ma, Mb, g2, beta)
    idx4a = i.reshape(NS, NCH_T // SUPG, SUPG, CHUNK)
    idx4b = i.reshape(NW, NCH_W // SUPG, SUPG, CHUNK)
    alphaf = _sc_softmax_den(alpha16.reshape(-1), idx4a, idx4b, zeros128)
    msum = _tc_msum(xjb, alphaf.reshape(E, 16))
    P = _sc_aggr(msum, i, zeros128)
    out, U, V = _tc_outuv(P, bias[None], W1[:D], W1[D:2 * D], b1[None])
    Ur, Vc = _gather2_128(U, i, V, j)
    e = _tc_edge(Ur, Vc, edge_fea, W1[2 * D:], W2, b2[None])
    return (out, e)


# speedup vs baseline: 1.2884x; 1.0366x over previous
import functools

import jax
import jax.numpy as jnp
from jax import lax
from jax.experimental import pallas as pl
from jax.experimental.pallas import tpu as pltpu
from jax.experimental.pallas import tpu_sc as plsc

N, E, D, DE, H = 10000, 320000, 128, 16, 4
HD = H * D
NC, NS, LL = 2, 16, 16
NW = NC * NS
CHUNK = 80
PER_W = E // NW
NCH_W = PER_W // CHUNK
PER_T = E // NS
NCH_T = PER_T // CHUNK
ROWS_T = 1000
EB = 512
NEB = E // EB
NB = 2000
NNB = N // NB

_mesh = plsc.VectorSubcoreMesh(core_axis_name="c", subcore_axis_name="s",
                               num_cores=NC, num_subcores=NS)


def _sp(x):
    return jnp.maximum(x, 0.0) + jnp.log1p(jnp.exp(-jnp.abs(x)))


def _silu(x):
    return x * (1.0 / (1.0 + jnp.exp(-x)))


def _alpha_body(ai_ref, aj_ref, ef_ref, wx_ref, we_ref,
                ma_ref, mb_ref, g2_ref, beta_ref, o_ref, xj_ref):
    et = jnp.dot(ef_ref[...], we_ref[...], preferred_element_type=jnp.float32)
    xi = _sp(jnp.dot(ai_ref[...].astype(jnp.bfloat16), wx_ref[...],
                     preferred_element_type=jnp.float32) + et)
    xj = _sp(jnp.dot(aj_ref[...].astype(jnp.bfloat16), wx_ref[...],
                     preferred_element_type=jnp.float32) + et)
    xj_ref[...] = xj.astype(jnp.bfloat16)
    draw = (jnp.dot(xi, ma_ref[...], preferred_element_type=jnp.float32)
            + jnp.dot(xj, mb_ref[...], preferred_element_type=jnp.float32))
    draw16 = jnp.concatenate([draw, jnp.zeros((EB, 16 - H), jnp.float32)],
                             axis=1)
    o_ref[...] = _sp(_sp(draw16) * g2_ref[...] + beta_ref[...])


def _tc_alpha(A_i, A_j, edge_fea, Wx_bf, We, Ma, Mb, g2, beta):
    return pl.pallas_call(
        _alpha_body,
        grid=(NEB,),
        in_specs=[pl.BlockSpec((EB, D), lambda b: (b, 0)),
                  pl.BlockSpec((EB, D), lambda b: (b, 0)),
                  pl.BlockSpec((EB, DE), lambda b: (b, 0)),
                  pl.BlockSpec((D, HD), lambda b: (0, 0)),
                  pl.BlockSpec((DE, HD), lambda b: (0, 0)),
                  pl.BlockSpec((HD, H), lambda b: (0, 0)),
                  pl.BlockSpec((HD, H), lambda b: (0, 0)),
                  pl.BlockSpec((1, 16), lambda b: (0, 0)),
                  pl.BlockSpec((1, 16), lambda b: (0, 0))],
        out_specs=[pl.BlockSpec((EB, 16), lambda b: (b, 0)),
                   pl.BlockSpec((EB, HD), lambda b: (b, 0))],
        out_shape=[jax.ShapeDtypeStruct((E, 16), jnp.float32),
                   jax.ShapeDtypeStruct((E, HD), jnp.bfloat16)],
    )(A_i, A_j, edge_fea, Wx_bf, We, Ma, Mb, g2, beta)


def _msum_body(xj_ref, af_ref, o_ref):
    xj = xj_ref[...].astype(jnp.float32)
    af = af_ref[...]
    acc = xj[:, 0:D] * af[:, 0:1]
    for h in range(1, H):
        acc = acc + xj[:, h * D:(h + 1) * D] * af[:, h:h + 1]
    o_ref[...] = acc * (1.0 / H)


def _tc_msum(xjb, alphaf):
    return pl.pallas_call(
        _msum_body,
        grid=(NEB,),
        in_specs=[pl.BlockSpec((EB, HD), lambda b: (b, 0)),
                  pl.BlockSpec((EB, 16), lambda b: (b, 0))],
        out_specs=pl.BlockSpec((EB, D), lambda b: (b, 0)),
        out_shape=jax.ShapeDtypeStruct((E, D), jnp.float32),
    )(xjb, alphaf)


def _outuv_body(p_ref, bias_ref, w1a_ref, w1b_ref, b1_ref,
                out_ref, u_ref, v_ref):
    s = p_ref[0] + p_ref[1] + bias_ref[...]
    out_ref[...] = s
    u_ref[...] = jnp.dot(s, w1a_ref[...],
                         preferred_element_type=jnp.float32) + b1_ref[...]
    v_ref[...] = jnp.dot(s, w1b_ref[...], preferred_element_type=jnp.float32)


def _tc_outuv(P, bias2d, W1a, W1b, b1_2d):
    return pl.pallas_call(
        _outuv_body,
        grid=(NNB,),
        in_specs=[pl.BlockSpec((NC, NB, D), lambda b: (0, b, 0)),
                  pl.BlockSpec((1, D), lambda b: (0, 0)),
                  pl.BlockSpec((D, D), lambda b: (0, 0)),
                  pl.BlockSpec((D, D), lambda b: (0, 0)),
                  pl.BlockSpec((1, D), lambda b: (0, 0))],
        out_specs=[pl.BlockSpec((NB, D), lambda b: (b, 0)),
                   pl.BlockSpec((NB, D), lambda b: (b, 0)),
                   pl.BlockSpec((NB, D), lambda b: (b, 0))],
        out_shape=[jax.ShapeDtypeStruct((N, D), jnp.float32)] * 3,
    )(P, bias2d, W1a, W1b, b1_2d)


def _edge_body(ur_ref, vc_ref, ef_ref, w1c_ref, w2_ref, b2_ref, o_ref):
    pre = (ur_ref[...] + vc_ref[...]
           + jnp.dot(ef_ref[...], w1c_ref[...],
                     preferred_element_type=jnp.float32))
    hh = _silu(pre)
    t = jnp.dot(hh, w2_ref[...], preferred_element_type=jnp.float32) + b2_ref[...]
    o_ref[...] = _silu(t)


def _tc_edge(Ur, Vc, edge_fea, W1c, W2, b2_2d):
    return pl.pallas_call(
        _edge_body,
        grid=(NEB,),
        in_specs=[pl.BlockSpec((EB, D), lambda b: (b, 0)),
                  pl.BlockSpec((EB, D), lambda b: (b, 0)),
                  pl.BlockSpec((EB, DE), lambda b: (b, 0)),
                  pl.BlockSpec((DE, D), lambda b: (0, 0)),
                  pl.BlockSpec((D, DE), lambda b: (0, 0)),
                  pl.BlockSpec((1, DE), lambda b: (0, 0))],
        out_specs=pl.BlockSpec((EB, DE), lambda b: (b, 0)),
        out_shape=jax.ShapeDtypeStruct((E, DE), jnp.float32),
    )(Ur, Vc, edge_fea, W1c, W2, b2_2d)


def _make_gather2(width, dtype=jnp.float32):
    nwin = PER_W // 5

    @functools.partial(
        pl.kernel,
        out_type=[jax.ShapeDtypeStruct((E, width), dtype)] * 2,
        mesh=_mesh,
        scratch_types=[
            pltpu.VMEM((nwin,), jnp.int32),
            pltpu.VMEM((CHUNK, width), dtype),
            pltpu.VMEM((CHUNK, width), dtype),
            pltpu.SemaphoreType.DMA,
            pltpu.SemaphoreType.DMA,
        ],
    )
    def k(tab_a, idx_a, tab_b, idx_b, out_a, out_b,
          idx_v, rows0, rows1, sem0, sem1):
        wid = lax.axis_index("s") * NC + lax.axis_index("c")
        base0 = wid * PER_W
        cpw = nwin // CHUNK
        rows = (rows0, rows1)
        sems = (sem0, sem1)

        for tab, idx, out in ((tab_a, idx_a, out_a), (tab_b, idx_b, out_b)):
            def wloop(w, _, tab=tab, idx=idx, out=out):
                pltpu.sync_copy(idx.at[pl.ds(base0 + w * nwin, nwin)], idx_v)
                descs = [None, None]
                descs[0] = pltpu.async_copy(
                    tab.at[idx_v.at[pl.ds(0, CHUNK)]], rows0, sem0)
                for ci in range(cpw):
                    b = ci & 1
                    descs[b].wait()
                    if ci + 1 < cpw:
                        nb = (ci + 1) & 1
                        descs[nb] = pltpu.async_copy(
                            tab.at[idx_v.at[pl.ds((ci + 1) * CHUNK, CHUNK)]],
                            rows[nb], sems[nb])
                    pltpu.sync_copy(
                        rows[b],
                        out.at[pl.ds(base0 + w * nwin + ci * CHUNK, CHUNK)])
                return _

            lax.fori_loop(0, PER_W // nwin, wloop, 0)

    return k


SUPG = 5
SUPR = SUPG * CHUNK


@functools.partial(
    pl.kernel,
    out_type=jax.ShapeDtypeStruct((E * 16,), jnp.float32),
    mesh=_mesh,
    scratch_types=[
        pltpu.VMEM((SUPG, CHUNK), jnp.int32),
        pltpu.VMEM((SUPR * 16,), jnp.float32),
        pltpu.VMEM((CHUNK, D), jnp.float32),
        pltpu.VMEM((CHUNK, D), jnp.float32),
        pltpu.SemaphoreType.DMA,
        pltpu.SemaphoreType.DMA,
        pltpu.VMEM_SHARED((N, D), jnp.float32),
    ],
)
def _sc_softmax_den(alpha_fl, idx4a, idx4b, zeros128, out,
                    idxv, av, ev0, ev1, sem0, sem1, den_sp):
    c = lax.axis_index("c")
    t = lax.axis_index("s")

    @pl.when(t < N // ROWS_T)
    def _():
        def zz(z, _):
            pltpu.sync_copy(zeros128.at[pl.ds(z * 200, 200)],
                            den_sp.at[pl.ds(t * ROWS_T + z * 200, 200)])
            return _

        lax.fori_loop(0, 5, zz, 0)

    pltpu.sync_copy(zeros128.at[pl.ds(0, CHUNK)], ev0)
    pltpu.sync_copy(zeros128.at[pl.ds(0, CHUNK)], ev1)
    plsc.subcore_barrier()
    evs = (ev0, ev1)
    sems = (sem0, sem1)

    def body_a(s, _):
        pltpu.sync_copy(
            alpha_fl.at[pl.ds((t * PER_T + s * SUPR) * 16, SUPR * 16)], av)
        pltpu.sync_copy(idx4a.at[t, s], idxv)
        descs = [None, None]
        for g in range(SUPG):
            b = g & 1
            if descs[b] is not None:
                descs[b].wait()
            ev = evs[b]
            for r in range(CHUNK):
                ev[r, pl.ds(0, 16)] = jnp.exp(
                    av[pl.ds((g * CHUNK + r) * 16, 16)])
            descs[b] = pltpu.async_copy(ev, den_sp.at[idxv.at[g]], sems[b],
                                        add=True)
        descs[0].wait()
        descs[1].wait()
        return _

    lax.fori_loop(0, NCH_T // SUPG, body_a, 0)
    plsc.subcore_barrier()

    wid = t * NC + c

    def body_b(s, _):
        base = wid * PER_W + s * SUPR
        pltpu.sync_copy(alpha_fl.at[pl.ds(base * 16, SUPR * 16)], av)

        pltpu.sync_copy(idx4b.at[wid, s], idxv)
        descs = [None, None]
        descs[0] = pltpu.async_copy(den_sp.at[idxv.at[0]], ev0, sem0)
        for g in range(SUPG):
            b = g & 1
            descs[b].wait()
            if g + 1 < SUPG:
                nb = (g + 1) & 1
                descs[nb] = pltpu.async_copy(den_sp.at[idxv.at[g + 1]],
                                             evs[nb], sems[nb])
            ev = evs[b]
            for r in range(CHUNK):
                rr = pl.ds((g * CHUNK + r) * 16, 16)
                av[rr] = jnp.exp(av[rr]) / (ev[r, pl.ds(0, 16)] + 1e-16)
        pltpu.sync_copy(av, out.at[pl.ds(base * 16, SUPR * 16)])
        return _

    lax.fori_loop(0, NCH_W // SUPG, body_b, 0)


@functools.partial(
    pl.kernel,
    out_type=jax.ShapeDtypeStruct((NC, N, D), jnp.float32),
    mesh=_mesh,
    scratch_types=[
        pltpu.VMEM((CHUNK,), jnp.int32),
        pltpu.VMEM((CHUNK, D), jnp.float32),
        pltpu.VMEM_SHARED((N, D), jnp.float32),
    ],
)
def _sc_aggr(msum, idx_i, zeros128, out, idxv, rows_v, acc_sp):
    c = lax.axis_index("c")
    t = lax.axis_index("s")

    @pl.when(t < N // ROWS_T)
    def _():
        pltpu.sync_copy(zeros128, acc_sp.at[pl.ds(t * ROWS_T, ROWS_T)])

    wid = t * NC + c
    plsc.subcore_barrier()

    base0 = wid * PER_W

    def body(ci, _):
        pltpu.sync_copy(idx_i.at[pl.ds(base0 + ci * CHUNK, CHUNK)], idxv)
        pltpu.sync_copy(msum.at[pl.ds(base0 + ci * CHUNK, CHUNK)], rows_v)
        pltpu.sync_copy(rows_v, acc_sp.at[idxv], add=True)
        return _

    lax.fori_loop(0, NCH_W, body, 0)
    plsc.subcore_barrier()

    @pl.when(t < N // ROWS_T)
    def _():
        pltpu.sync_copy(acc_sp.at[pl.ds(t * ROWS_T, ROWS_T)],
                        out.at[c, pl.ds(t * ROWS_T, ROWS_T)])


_gather2_128 = _make_gather2(D)


def kernel(atom_fea, edge_idx, edge_fea, batch, distance, edge_vec,
           W, att, bias, bn_gamma, bn_beta, W1, b1, W2, b2):
    i = edge_idx[0]
    j = edge_idx[1]
    Wx = W[:D]
    We = W[D:]
    att1 = att[0, :, :D]
    att2 = att[0, :, D:]
    dd = jnp.arange(HD)
    Ma = jnp.zeros((HD, H), jnp.float32).at[dd, dd // D].set(att1.reshape(-1))
    Mb = jnp.zeros((HD, H), jnp.float32).at[dd, dd // D].set(att2.reshape(-1))
    g2 = jnp.zeros((1, 16), jnp.float32).at[0, :H].set(
        bn_gamma / jnp.sqrt(1.0 + 1e-5))
    beta = jnp.zeros((1, 16), jnp.float32).at[0, :H].set(bn_beta)
    zeros128 = jnp.zeros((ROWS_T, D), jnp.float32)
    del batch, distance, edge_vec

    Wx_bf = Wx.astype(jnp.bfloat16)
    A_i, A_j = _gather2_128(atom_fea, i, atom_fea, j)
    alpha16, xjb = _tc_alpha(A_i, A_j, edge_fea, Wx_bf, We, Ma, Mb, g2, beta)
    idx4a = i.reshape(NS, NCH_T // SUPG, SUPG, CHUNK)
    idx4b = i.reshape(NW, NCH_W // SUPG, SUPG, CHUNK)
    alphaf = _sc_softmax_den(alpha16.reshape(-1), idx4a, idx4b, zeros128)
    msum = _tc_msum(xjb, alphaf.reshape(E, 16))
    P = _sc_aggr(msum, i, zeros128)
    out, U, V = _tc_outuv(P, bias[None], W1[:D], W1[D:2 * D], b1[None])
    Ur, Vc = _gather2_128(U, i, V, j)
    e = _tc_edge(Ur, Vc, edge_fea, W1[2 * D:], W2, b2[None])
    return (out, e)


# speedup vs baseline: 1.3166x; 1.0219x over previous
import functools

import jax
import jax.numpy as jnp
from jax import lax
from jax.experimental import pallas as pl
from jax.experimental.pallas import tpu as pltpu
from jax.experimental.pallas import tpu_sc as plsc

N, E, D, DE, H = 10000, 320000, 128, 16, 4
HD = H * D
NC, NS, LL = 2, 16, 16
NW = NC * NS
CHUNK = 80
PER_W = E // NW
NCH_W = PER_W // CHUNK
PER_T = E // NS
NCH_T = PER_T // CHUNK
ROWS_T = 1000
EB = 512
NEB = E // EB
NB = 2000
NNB = N // NB

_mesh = plsc.VectorSubcoreMesh(core_axis_name="c", subcore_axis_name="s",
                               num_cores=NC, num_subcores=NS)


def _sp(x):
    return jnp.maximum(x, 0.0) + jnp.log1p(jnp.exp(-jnp.abs(x)))


def _silu(x):
    return x * (1.0 / (1.0 + jnp.exp(-x)))


def _alpha_body(ai_ref, aj_ref, ef_ref, wx_ref, we_ref,
                ma_ref, mb_ref, g2_ref, beta_ref, o_ref, xj_ref):
    et = jnp.dot(ef_ref[...], we_ref[...], preferred_element_type=jnp.float32)
    xi = _sp(jnp.dot(ai_ref[...].astype(jnp.bfloat16), wx_ref[...],
                     preferred_element_type=jnp.float32) + et)
    xj = _sp(jnp.dot(aj_ref[...].astype(jnp.bfloat16), wx_ref[...],
                     preferred_element_type=jnp.float32) + et)
    xj_ref[...] = xj.astype(jnp.bfloat16)
    draw = (jnp.dot(xi, ma_ref[...], preferred_element_type=jnp.float32)
            + jnp.dot(xj, mb_ref[...], preferred_element_type=jnp.float32))
    draw16 = jnp.concatenate([draw, jnp.zeros((EB, 16 - H), jnp.float32)],
                             axis=1)
    o_ref[...] = _sp(_sp(draw16) * g2_ref[...] + beta_ref[...])


def _tc_alpha(A_i, A_j, edge_fea, Wx_bf, We, Ma, Mb, g2, beta):
    return pl.pallas_call(
        _alpha_body,
        grid=(NEB,),
        in_specs=[pl.BlockSpec((EB, D), lambda b: (b, 0)),
                  pl.BlockSpec((EB, D), lambda b: (b, 0)),
                  pl.BlockSpec((EB, DE), lambda b: (b, 0)),
                  pl.BlockSpec((D, HD), lambda b: (0, 0)),
                  pl.BlockSpec((DE, HD), lambda b: (0, 0)),
                  pl.BlockSpec((HD, H), lambda b: (0, 0)),
                  pl.BlockSpec((HD, H), lambda b: (0, 0)),
                  pl.BlockSpec((1, 16), lambda b: (0, 0)),
                  pl.BlockSpec((1, 16), lambda b: (0, 0))],
        out_specs=[pl.BlockSpec((EB, 16), lambda b: (b, 0)),
                   pl.BlockSpec((EB, HD), lambda b: (b, 0))],
        out_shape=[jax.ShapeDtypeStruct((E, 16), jnp.float32),
                   jax.ShapeDtypeStruct((E, HD), jnp.bfloat16)],
    )(A_i, A_j, edge_fea, Wx_bf, We, Ma, Mb, g2, beta)


def _msum_body(xj_ref, af_ref, o_ref):
    xj = xj_ref[...].astype(jnp.float32)
    af = af_ref[...]
    acc = xj[:, 0:D] * af[:, 0:1]
    for h in range(1, H):
        acc = acc + xj[:, h * D:(h + 1) * D] * af[:, h:h + 1]
    o_ref[...] = acc * (1.0 / H)


def _tc_msum(xjb, alphaf):
    return pl.pallas_call(
        _msum_body,
        grid=(NEB,),
        in_specs=[pl.BlockSpec((EB, HD), lambda b: (b, 0)),
                  pl.BlockSpec((EB, 16), lambda b: (b, 0))],
        out_specs=pl.BlockSpec((EB, D), lambda b: (b, 0)),
        out_shape=jax.ShapeDtypeStruct((E, D), jnp.float32),
    )(xjb, alphaf)


def _outuv_body(p_ref, bias_ref, w1a_ref, w1b_ref, b1_ref,
                out_ref, u_ref, v_ref):
    s = p_ref[0] + p_ref[1] + bias_ref[...]
    out_ref[...] = s
    u_ref[...] = jnp.dot(s, w1a_ref[...],
                         preferred_element_type=jnp.float32) + b1_ref[...]
    v_ref[...] = jnp.dot(s, w1b_ref[...], preferred_element_type=jnp.float32)


def _tc_outuv(P, bias2d, W1a, W1b, b1_2d):
    return pl.pallas_call(
        _outuv_body,
        grid=(NNB,),
        in_specs=[pl.BlockSpec((NC, NB, D), lambda b: (0, b, 0)),
                  pl.BlockSpec((1, D), lambda b: (0, 0)),
                  pl.BlockSpec((D, D), lambda b: (0, 0)),
                  pl.BlockSpec((D, D), lambda b: (0, 0)),
                  pl.BlockSpec((1, D), lambda b: (0, 0))],
        out_specs=[pl.BlockSpec((NB, D), lambda b: (b, 0)),
                   pl.BlockSpec((NB, D), lambda b: (b, 0)),
                   pl.BlockSpec((NB, D), lambda b: (b, 0))],
        out_shape=[jax.ShapeDtypeStruct((N, D), jnp.float32)] * 3,
    )(P, bias2d, W1a, W1b, b1_2d)


def _edge_body(ur_ref, vc_ref, ef_ref, w1c_ref, w2_ref, b2_ref, o_ref):
    pre = (ur_ref[...] + vc_ref[...]
           + jnp.dot(ef_ref[...], w1c_ref[...],
                     preferred_element_type=jnp.float32))
    hh = _silu(pre)
    t = jnp.dot(hh, w2_ref[...], preferred_element_type=jnp.float32) + b2_ref[...]
    o_ref[...] = _silu(t)


def _tc_edge(Ur, Vc, edge_fea, W1c, W2, b2_2d):
    return pl.pallas_call(
        _edge_body,
        grid=(NEB,),
        in_specs=[pl.BlockSpec((EB, D), lambda b: (b, 0)),
                  pl.BlockSpec((EB, D), lambda b: (b, 0)),
                  pl.BlockSpec((EB, DE), lambda b: (b, 0)),
                  pl.BlockSpec((DE, D), lambda b: (0, 0)),
                  pl.BlockSpec((D, DE), lambda b: (0, 0)),
                  pl.BlockSpec((1, DE), lambda b: (0, 0))],
        out_specs=pl.BlockSpec((EB, DE), lambda b: (b, 0)),
        out_shape=jax.ShapeDtypeStruct((E, DE), jnp.float32),
    )(Ur, Vc, edge_fea, W1c, W2, b2_2d)


def _make_gather2(width, dtype=jnp.float32):
    nwin = PER_W // 5

    @functools.partial(
        pl.kernel,
        out_type=[jax.ShapeDtypeStruct((E, width), dtype)] * 2,
        mesh=_mesh,
        scratch_types=[
            pltpu.VMEM((nwin,), jnp.int32),
            pltpu.VMEM((CHUNK, width), dtype),
            pltpu.VMEM((CHUNK, width), dtype),
            pltpu.SemaphoreType.DMA,
            pltpu.SemaphoreType.DMA,
        ],
    )
    def k(tab_a, idx_a, tab_b, idx_b, out_a, out_b,
          idx_v, rows0, rows1, sem0, sem1):
        wid = lax.axis_index("s") * NC + lax.axis_index("c")
        base0 = wid * PER_W
        cpw = nwin // CHUNK
        rows = (rows0, rows1)
        sems = (sem0, sem1)

        for tab, idx, out in ((tab_a, idx_a, out_a), (tab_b, idx_b, out_b)):
            def wloop(w, _, tab=tab, idx=idx, out=out):
                pltpu.sync_copy(idx.at[pl.ds(base0 + w * nwin, nwin)], idx_v)
                descs = [None, None]
                descs[0] = pltpu.async_copy(
                    tab.at[idx_v.at[pl.ds(0, CHUNK)]], rows0, sem0)
                for ci in range(cpw):
                    b = ci & 1
                    descs[b].wait()
                    if ci + 1 < cpw:
                        nb = (ci + 1) & 1
                        descs[nb] = pltpu.async_copy(
                            tab.at[idx_v.at[pl.ds((ci + 1) * CHUNK, CHUNK)]],
                            rows[nb], sems[nb])
                    pltpu.sync_copy(
                        rows[b],
                        out.at[pl.ds(base0 + w * nwin + ci * CHUNK, CHUNK)])
                return _

            lax.fori_loop(0, PER_W // nwin, wloop, 0)

    return k


SUPG = 5
SUPR = SUPG * CHUNK


@functools.partial(
    pl.kernel,
    out_type=jax.ShapeDtypeStruct((E * 16,), jnp.float32),
    mesh=_mesh,
    scratch_types=[
        pltpu.VMEM((SUPG, CHUNK), jnp.int32),
        pltpu.VMEM((SUPR * 16,), jnp.float32),
        pltpu.VMEM((CHUNK, D), jnp.float32),
        pltpu.VMEM((CHUNK, D), jnp.float32),
        pltpu.SemaphoreType.DMA,
        pltpu.SemaphoreType.DMA,
        pltpu.VMEM_SHARED((N, D), jnp.float32),
    ],
)
def _sc_softmax_den(alpha_fl, idx4a, idx4b, zeros128, out,
                    idxv, av, ev0, ev1, sem0, sem1, den_sp):
    c = lax.axis_index("c")
    t = lax.axis_index("s")

    @pl.when(t < N // ROWS_T)
    def _():
        def zz(z, _):
            pltpu.sync_copy(zeros128.at[pl.ds(z * 200, 200)],
                            den_sp.at[pl.ds(t * ROWS_T + z * 200, 200)])
            return _

        lax.fori_loop(0, 5, zz, 0)

    pltpu.sync_copy(zeros128.at[pl.ds(0, CHUNK)], ev0)
    pltpu.sync_copy(zeros128.at[pl.ds(0, CHUNK)], ev1)
    plsc.subcore_barrier()
    evs = (ev0, ev1)
    sems = (sem0, sem1)

    def body_a(s, _):
        pltpu.sync_copy(
            alpha_fl.at[pl.ds((t * PER_T + s * SUPR) * 16, SUPR * 16)], av)
        pltpu.sync_copy(idx4a.at[t, s], idxv)
        descs = [None, None]
        for g in range(SUPG):
            b = g & 1
            if descs[b] is not None:
                descs[b].wait()
            ev = evs[b]
            for r in range(CHUNK):
                ev[r, pl.ds(0, 16)] = jnp.exp(
                    av[pl.ds((g * CHUNK + r) * 16, 16)])
            descs[b] = pltpu.async_copy(ev, den_sp.at[idxv.at[g]], sems[b],
                                        add=True)
        descs[0].wait()
        descs[1].wait()
        return _

    lax.fori_loop(0, NCH_T // SUPG, body_a, 0)
    plsc.subcore_barrier()

    wid = t * NC + c

    def body_b(s, _):
        base = wid * PER_W + s * SUPR
        pltpu.sync_copy(alpha_fl.at[pl.ds(base * 16, SUPR * 16)], av)

        pltpu.sync_copy(idx4b.at[wid, s], idxv)
        descs = [None, None]
        descs[0] = pltpu.async_copy(den_sp.at[idxv.at[0]], ev0, sem0)
        for g in range(SUPG):
            b = g & 1
            descs[b].wait()
            if g + 1 < SUPG:
                nb = (g + 1) & 1
                descs[nb] = pltpu.async_copy(den_sp.at[idxv.at[g + 1]],
                                             evs[nb], sems[nb])
            ev = evs[b]
            for r in range(CHUNK):
                rr = pl.ds((g * CHUNK + r) * 16, 16)
                av[rr] = jnp.exp(av[rr]) / (ev[r, pl.ds(0, 16)] + 1e-16)
        pltpu.sync_copy(av, out.at[pl.ds(base * 16, SUPR * 16)])
        return _

    lax.fori_loop(0, NCH_W // SUPG, body_b, 0)


@functools.partial(
    pl.kernel,
    out_type=jax.ShapeDtypeStruct((NC, N, D), jnp.float32),
    mesh=_mesh,
    scratch_types=[
        pltpu.VMEM((SUPG, CHUNK), jnp.int32),
        pltpu.VMEM((CHUNK, D), jnp.float32),
        pltpu.VMEM((CHUNK, D), jnp.float32),
        pltpu.SemaphoreType.DMA,
        pltpu.SemaphoreType.DMA,
        pltpu.VMEM_SHARED((N, D), jnp.float32),
    ],
)
def _sc_aggr(msum, idx4b, zeros128, out, idxv, rows0, rows1, sem0, sem1,
             acc_sp):
    c = lax.axis_index("c")
    t = lax.axis_index("s")

    @pl.when(t < N // ROWS_T)
    def _():
        pltpu.sync_copy(zeros128, acc_sp.at[pl.ds(t * ROWS_T, ROWS_T)])

    wid = t * NC + c
    plsc.subcore_barrier()

    base0 = wid * PER_W
    rows = (rows0, rows1)
    sems = (sem0, sem1)

    def body(s, _):
        pltpu.sync_copy(idx4b.at[wid, s], idxv)
        descs = [None, None]
        for g in range(SUPG):
            b = g & 1
            if descs[b] is not None:
                descs[b].wait()
            pltpu.sync_copy(
                msum.at[pl.ds(base0 + (s * SUPG + g) * CHUNK, CHUNK)],
                rows[b])
            descs[b] = pltpu.async_copy(rows[b], acc_sp.at[idxv.at[g]],
                                        sems[b], add=True)
        descs[0].wait()
        descs[1].wait()
        return _

    lax.fori_loop(0, NCH_W // SUPG, body, 0)
    plsc.subcore_barrier()

    @pl.when(t < N // ROWS_T)
    def _():
        pltpu.sync_copy(acc_sp.at[pl.ds(t * ROWS_T, ROWS_T)],
                        out.at[c, pl.ds(t * ROWS_T, ROWS_T)])


_gather2_128 = _make_gather2(D)


def kernel(atom_fea, edge_idx, edge_fea, batch, distance, edge_vec,
           W, att, bias, bn_gamma, bn_beta, W1, b1, W2, b2):
    i = edge_idx[0]
    j = edge_idx[1]
    Wx = W[:D]
    We = W[D:]
    att1 = att[0, :, :D]
    att2 = att[0, :, D:]
    dd = jnp.arange(HD)
    Ma = jnp.zeros((HD, H), jnp.float32).at[dd, dd // D].set(att1.reshape(-1))
    Mb = jnp.zeros((HD, H), jnp.float32).at[dd, dd // D].set(att2.reshape(-1))
    g2 = jnp.zeros((1, 16), jnp.float32).at[0, :H].set(
        bn_gamma / jnp.sqrt(1.0 + 1e-5))
    beta = jnp.zeros((1, 16), jnp.float32).at[0, :H].set(bn_beta)
    zeros128 = jnp.zeros((ROWS_T, D), jnp.float32)
    del batch, distance, edge_vec

    Wx_bf = Wx.astype(jnp.bfloat16)
    A_i, A_j = _gather2_128(atom_fea, i, atom_fea, j)
    alpha16, xjb = _tc_alpha(A_i, A_j, edge_fea, Wx_bf, We, Ma, Mb, g2, beta)
    idx4a = i.reshape(NS, NCH_T // SUPG, SUPG, CHUNK)
    idx4b = i.reshape(NW, NCH_W // SUPG, SUPG, CHUNK)
    alphaf = _sc_softmax_den(alpha16.reshape(-1), idx4a, idx4b, zeros128)
    msum = _tc_msum(xjb, alphaf.reshape(E, 16))
    P = _sc_aggr(msum, idx4b, zeros128)
    out, U, V = _tc_outuv(P, bias[None], W1[:D], W1[D:2 * D], b1[None])
    Ur, Vc = _gather2_128(U, i, V, j)
    e = _tc_edge(Ur, Vc, edge_fea, W1[2 * D:], W2, b2[None])
    return (out, e)
